# baseline (speedup 1.0000x reference)
import functools

import jax
import jax.numpy as jnp
from jax.experimental import pallas as pl
from jax.experimental.pallas import tpu as pltpu

ALPHA = 32.0


def _fused_kernel(x_ref, wb_ref, b_ref, wg_ref, a_ref, bl_ref, o_ref,
                  *, n_exp, rank, scaling):
    x = x_ref[...]
    tn = x.shape[0]

    base = jax.lax.dot_general(
        x, wb_ref[...], (((1,), (1,)), ((), ())),
        preferred_element_type=jnp.float32)

    logits = jax.lax.dot_general(
        wg_ref[...], x, (((1,), (1,)), ((), ())),
        preferred_element_type=jnp.float32)
    idx = jax.lax.broadcasted_iota(jnp.int32, (n_exp, tn), 0)
    m1 = jnp.max(logits, axis=0, keepdims=True)
    i1 = jnp.min(jnp.where(logits == m1, idx, n_exp), axis=0, keepdims=True)
    masked = jnp.where(idx == i1, -jnp.inf, logits)
    m2 = jnp.max(masked, axis=0, keepdims=True)
    i2 = jnp.min(jnp.where(masked == m2, idx, n_exp), axis=0, keepdims=True)
    e21 = jnp.exp(m2 - m1)
    inv = scaling / (1.0 + e21)
    i1c = i1.reshape(tn, 1)
    i2c = i2.reshape(tn, 1)
    w1c = inv.reshape(tn, 1)
    w2c = (inv * e21).reshape(tn, 1)
    lane_e = jax.lax.broadcasted_iota(jnp.int32, (tn, n_exp * rank), 1) // rank
    w_exp = (w1c * (lane_e == i1c).astype(jnp.float32)
             + w2c * (lane_e == i2c).astype(jnp.float32))

    h = jax.lax.dot_general(
        x, a_ref[...], (((1,), (1,)), ((), ())),
        preferred_element_type=jnp.float32)
    hw = h * w_exp

    moe = jax.lax.dot_general(
        hw, bl_ref[...], (((1,), (0,)), ((), ())),
        preferred_element_type=jnp.float32)
    o_ref[...] = base + moe + b_ref[...]


def kernel(inputs, W_base, b_base, W_gate, lora_A, lora_B):
    b, s, d = inputs.shape
    out_f = W_base.shape[0]
    n_exp, rank = lora_A.shape[0], lora_A.shape[1]
    scaling = ALPHA / rank
    n = b * s

    flat = inputs.reshape(n, d)
    a_all = lora_A.reshape(n_exp * rank, d)
    b_all = lora_B.transpose(0, 2, 1).reshape(n_exp * rank, out_f)
    bias2 = b_base.reshape(1, out_f)

    tn = 1024
    while n % tn:
        tn //= 2
    grid = (n // tn,)

    out = pl.pallas_call(
        functools.partial(_fused_kernel, n_exp=n_exp, rank=rank, scaling=scaling),
        grid=grid,
        in_specs=[
            pl.BlockSpec((tn, d), lambda i: (i, 0)),
            pl.BlockSpec((out_f, d), lambda i: (0, 0)),
            pl.BlockSpec((1, out_f), lambda i: (0, 0)),
            pl.BlockSpec((n_exp, d), lambda i: (0, 0)),
            pl.BlockSpec((n_exp * rank, d), lambda i: (0, 0)),
            pl.BlockSpec((n_exp * rank, out_f), lambda i: (0, 0)),
        ],
        out_specs=pl.BlockSpec((tn, out_f), lambda i: (i, 0)),
        out_shape=jax.ShapeDtypeStruct((n, out_f), jnp.float32),
    )(flat, W_base, bias2, W_gate, a_all, b_all)

    return out.reshape(b, s, out_f)


# speedup vs baseline: 1.0019x; 1.0019x over previous
import functools

import jax
import jax.numpy as jnp
from jax.experimental import pallas as pl
from jax.experimental.pallas import tpu as pltpu

ALPHA = 32.0


def _fused_kernel(x_ref, wb_ref, b_ref, wg_ref, a_ref, bl_ref, o_ref,
                  *, n_exp, rank, scaling):
    x = x_ref[...]
    tn = x.shape[0]

    base = jax.lax.dot_general(
        x, wb_ref[...], (((1,), (1,)), ((), ())),
        preferred_element_type=jnp.float32)

    logits = jax.lax.dot_general(
        x, wg_ref[...], (((1,), (1,)), ((), ())),
        preferred_element_type=jnp.float32)
    idx = jax.lax.broadcasted_iota(jnp.int32, (tn, n_exp), 1)
    m1 = jnp.max(logits, axis=1, keepdims=True)
    i1 = jnp.min(jnp.where(logits == m1, idx, n_exp), axis=1, keepdims=True)
    masked = jnp.where(idx == i1, -jnp.inf, logits)
    m2 = jnp.max(masked, axis=1, keepdims=True)
    i2 = jnp.min(jnp.where(masked == m2, idx, n_exp), axis=1, keepdims=True)
    e21 = jnp.exp(m2 - m1)
    inv = scaling / (1.0 + e21)
    nl = n_exp * rank + 8
    lane = jax.lax.broadcasted_iota(jnp.int32, (tn, nl), 1)
    lane_e = lane // rank
    w_exp = (inv * (lane_e == i1).astype(jnp.float32)
             + (inv * e21) * (lane_e == i2).astype(jnp.float32))

    h = jax.lax.dot_general(
        x, a_ref[...], (((1,), (1,)), ((), ())),
        preferred_element_type=jnp.float32)
    hw = jnp.where(lane >= n_exp * rank, 0.125, h * w_exp)

    moe = jax.lax.dot_general(
        hw, bl_ref[...], (((1,), (0,)), ((), ())),
        preferred_element_type=jnp.float32)
    o_ref[...] = base + moe


def kernel(inputs, W_base, b_base, W_gate, lora_A, lora_B):
    b, s, d = inputs.shape
    out_f = W_base.shape[0]
    n_exp, rank = lora_A.shape[0], lora_A.shape[1]
    scaling = ALPHA / rank
    n = b * s

    flat = inputs.reshape(n, d)
    a_all = jnp.concatenate(
        [lora_A.reshape(n_exp * rank, d),
         jnp.zeros((8, d), jnp.float32)], axis=0)
    b_all = jnp.concatenate(
        [lora_B.transpose(0, 2, 1).reshape(n_exp * rank, out_f),
         jnp.tile(b_base.reshape(1, out_f), (8, 1))], axis=0)
    bias2 = b_base.reshape(1, out_f)

    tn = 1024
    while n % tn:
        tn //= 2
    grid = (n // tn,)

    out = pl.pallas_call(
        functools.partial(_fused_kernel, n_exp=n_exp, rank=rank, scaling=scaling),
        grid=grid,
        in_specs=[
            pl.BlockSpec((tn, d), lambda i: (i, 0)),
            pl.BlockSpec((out_f, d), lambda i: (0, 0)),
            pl.BlockSpec((1, out_f), lambda i: (0, 0)),
            pl.BlockSpec((n_exp, d), lambda i: (0, 0)),
            pl.BlockSpec((n_exp * rank + 8, d), lambda i: (0, 0)),
            pl.BlockSpec((n_exp * rank + 8, out_f), lambda i: (0, 0)),
        ],
        out_specs=pl.BlockSpec((tn, out_f), lambda i: (i, 0)),
        out_shape=jax.ShapeDtypeStruct((n, out_f), jnp.float32),
    )(flat, W_base, bias2, W_gate, a_all, b_all)

    return out.reshape(b, s, out_f)


# speedup vs baseline: 1.0753x; 1.0733x over previous
import functools

import jax
import jax.numpy as jnp
from jax.experimental import pallas as pl

ALPHA = 32.0


def _fused_kernel(x_ref, wb_ref, b_ref, wg_ref, a_ref, bl_ref, o_ref,
                  *, n_exp, rank, scaling):
    x = x_ref[...]
    tn = x.shape[0]

    base = jax.lax.dot_general(
        x, wb_ref[...], (((1,), (1,)), ((), ())),
        preferred_element_type=jnp.float32)

    logits = jax.lax.dot_general(
        x, wg_ref[...], (((1,), (1,)), ((), ())),
        preferred_element_type=jnp.float32)
    idx = jax.lax.broadcasted_iota(jnp.int32, (tn, n_exp), 1)
    m1 = jnp.max(logits, axis=1, keepdims=True)
    i1 = jnp.min(jnp.where(logits == m1, idx, n_exp), axis=1, keepdims=True)
    masked = jnp.where(idx == i1, -jnp.inf, logits)
    m2 = jnp.max(masked, axis=1, keepdims=True)
    i2 = jnp.min(jnp.where(masked == m2, idx, n_exp), axis=1, keepdims=True)
    e21 = jnp.exp(m2 - m1)
    inv = scaling / (1.0 + e21)
    lane_e = jax.lax.broadcasted_iota(jnp.int32, (tn, n_exp * rank), 1) // rank
    w_exp = (inv * (lane_e == i1).astype(jnp.float32)
             + (inv * e21) * (lane_e == i2).astype(jnp.float32))

    h = jax.lax.dot_general(
        x, a_ref[...], (((1,), (1,)), ((), ())),
        preferred_element_type=jnp.float32)
    hw = h * w_exp

    moe = jax.lax.dot_general(
        hw, bl_ref[...], (((1,), (0,)), ((), ())),
        preferred_element_type=jnp.float32)
    o_ref[...] = base + moe + b_ref[...]


def kernel(inputs, W_base, b_base, W_gate, lora_A, lora_B):
    b, s, d = inputs.shape
    out_f = W_base.shape[0]
    n_exp, rank = lora_A.shape[0], lora_A.shape[1]
    scaling = ALPHA / rank
    n = b * s

    flat = inputs.reshape(n, d)
    a_all = lora_A.reshape(n_exp * rank, d)
    b_all = lora_B.transpose(0, 2, 1).reshape(n_exp * rank, out_f)
    bias2 = b_base.reshape(1, out_f)

    tn = 1024
    while n % tn:
        tn //= 2
    grid = (n // tn,)

    out = pl.pallas_call(
        functools.partial(_fused_kernel, n_exp=n_exp, rank=rank, scaling=scaling),
        grid=grid,
        in_specs=[
            pl.BlockSpec((tn, d), lambda i: (i, 0)),
            pl.BlockSpec((out_f, d), lambda i: (0, 0)),
            pl.BlockSpec((1, out_f), lambda i: (0, 0)),
            pl.BlockSpec((n_exp, d), lambda i: (0, 0)),
            pl.BlockSpec((n_exp * rank, d), lambda i: (0, 0)),
            pl.BlockSpec((n_exp * rank, out_f), lambda i: (0, 0)),
        ],
        out_specs=pl.BlockSpec((tn, out_f), lambda i: (i, 0)),
        out_shape=jax.ShapeDtypeStruct((n, out_f), jnp.float32),
    )(flat, W_base, bias2, W_gate, a_all, b_all)

    return out.reshape(b, s, out_f)
